# baseline (speedup 1.0000x reference)
import functools

import jax
import jax.numpy as jnp
from jax import lax
from jax.experimental import pallas as pl
from jax.experimental.pallas import tpu as pltpu
from jax.experimental.pallas import tpu_sc as plsc

D = 256
NKX = D // 2 + 1
NIMG = 32
NPTS = NIMG * D * NKX
NZY = D * D
SENT = 1 << 24
QS = 1024
QROWS = 64
NW = 32
CHUNK = NPTS // NW
SUB = 4
SCH = CHUNK // SUB
LANES = 16
LBINS = (QS + 1) * LANES
LBINS_P = 16416
RS = SCH + QS + 16
TBLN = 1040
BRUN = 64

REC_W = 4
REGW = RS * REC_W
BRECW = NW * SUB * REGW + 1024

_SC_PARAMS = pltpu.CompilerParams(needs_layout_passes=False)


def _mesh():
    return plsc.VectorSubcoreMesh(core_axis_name="c", subcore_axis_name="s")


def _wid():
    return lax.axis_index("s") * 2 + lax.axis_index("c")


def _prep_body(fr_ref, fi_ref, ctf_ref, neg_ref, syky_ref, sxkx_ref,
               vr_ref, vi_ref, cc_ref):
    fr = fr_ref[0]
    fi = fi_ref[0]
    ctf = ctf_ref[0]
    ph = (-2.0 * jnp.pi) * (syky_ref[0, 0][:, None] + sxkx_ref[0, 0][None, :])
    c = jnp.cos(ph)
    s = jnp.sin(ph)
    pr = (fr * c - fi * s) * ctf
    pi = (fr * s + fi * c) * ctf
    sign = 1.0 - 2.0 * neg_ref[0]
    vr_ref[0] = pr
    vi_ref[0] = pi * sign
    cc_ref[0] = ctf * ctf


def _prep(fr, fi, ctf, negf, syky, sxkx):
    blk = pl.BlockSpec((1, D, NKX), lambda b: (b, 0, 0))
    sblk = lambda n: pl.BlockSpec((1, 1, n), lambda b: (b, 0, 0))
    return pl.pallas_call(
        _prep_body,
        grid=(NIMG,),
        in_specs=[blk, blk, blk, blk, sblk(D), sblk(NKX)],
        out_specs=[blk, blk, blk],
        out_shape=[jax.ShapeDtypeStruct((NIMG, D, NKX), jnp.float32)] * 3,
    )(fr, fi, ctf, negf, syky.reshape(NIMG, 1, D), sxkx.reshape(NIMG, 1, NKX))


def _sortflush_body(idxf, vr, vi, cc, brec, offt, lent,
                    ibuf, vbr, vbi, vbc, lhist, loff, ptab, ltab,
                    sorted_buf, sem):
    wid = _wid()
    lane = lax.iota(jnp.int32, LANES)
    zero16 = jnp.zeros((LANES,), jnp.int32)
    one16 = jnp.ones((LANES,), jnp.int32)

    def subchunk(s, carry):
        base = wid * CHUNK + s * SCH
        region = (wid * SUB + s) * RS

        pltpu.sync_copy(idxf.at[pl.ds(base, SCH)], ibuf)
        pltpu.sync_copy(vr.at[pl.ds(base, SCH)], vbr)
        pltpu.sync_copy(vi.at[pl.ds(base, SCH)], vbi)
        pltpu.sync_copy(cc.at[pl.ds(base, SCH)], vbc)

        def zero(i, c2):
            lhist[pl.ds(i * 16, 16)] = zero16
            return c2

        lax.fori_loop(0, LBINS_P // 16, zero, 0)

        def zero2(i, c2):
            ltab[pl.ds(i * 16, 16)] = zero16
            return c2

        lax.fori_loop(0, TBLN // 16, zero2, 0)

        def h1(v, c2):
            iv = ibuf[pl.ds(v * 16, 16)]
            q = lax.shift_right_logical(iv, 14)
            plsc.addupdate_scatter(lhist, [q * LANES + lane], one16)
            plsc.addupdate_scatter(ltab, [q], one16)
            return c2

        lax.fori_loop(0, SCH // 16, h1, 0)

        def pfxa(g, run):
            tot = ltab[pl.ds(g * 16, 16)]
            ptot = (tot + 1) & jnp.int32(~1)
            cs = plsc.cumsum(ptot)
            ptab[pl.ds(g * 16, 16)] = (cs - ptot) + jnp.full(
                (LANES,), run + region, jnp.int32)
            return run + jnp.max(cs)

        lax.fori_loop(0, (QS + 16) // 16, pfxa, jnp.int32(0))

        pltpu.sync_copy(ptab, offt.at[wid, s])
        pltpu.sync_copy(ltab, lent.at[wid, s])

        def pfxb(q, c2):
            h = lhist[pl.ds(q * 16, 16)]
            excl = plsc.cumsum(h) - h
            qb = plsc.load_gather(ptab, [jnp.full((LANES,), q, jnp.int32)])
            loff[pl.ds(q * 16, 16)] = excl + qb - jnp.full(
                (LANES,), region, jnp.int32)
            return c2

        lax.fori_loop(0, QS + 1, pfxb, 0)

        def p2(v, c2):
            iv = ibuf[pl.ds(v * 16, 16)]
            binc = lax.shift_right_logical(iv, 14) * LANES + lane
            pos = plsc.load_gather(loff, [binc])
            plsc.store_scatter(loff, [binc], pos + 1)
            widx = pos * REC_W
            rel = iv & jnp.int32(16383)
            plsc.store_scatter(sorted_buf, [widx],
                               plsc.bitcast(rel, jnp.float32))
            plsc.store_scatter(sorted_buf, [widx + 1], vbr[pl.ds(v * 16, 16)])
            plsc.store_scatter(sorted_buf, [widx + 2], vbi[pl.ds(v * 16, 16)])
            plsc.store_scatter(sorted_buf, [widx + 3], vbc[pl.ds(v * 16, 16)])
            return c2

        lax.fori_loop(0, SCH // 16, p2, 0)

        pltpu.sync_copy(sorted_buf, brec.at[pl.ds(region * REC_W, REGW)])
        return carry

    lax.fori_loop(0, SUB, subchunk, 0)


def _sortflush(idxf, vr, vi, cc):
    k = functools.partial(
        pl.kernel,
        mesh=_mesh(),
        compiler_params=_SC_PARAMS,
        out_type=[
            jax.ShapeDtypeStruct((BRECW,), jnp.float32),
            jax.ShapeDtypeStruct((NW, SUB, TBLN), jnp.int32),
            jax.ShapeDtypeStruct((NW, SUB, TBLN), jnp.int32),
        ],
        scratch_types=[
            pltpu.VMEM((SCH,), jnp.int32),
            pltpu.VMEM((SCH,), jnp.float32),
            pltpu.VMEM((SCH,), jnp.float32),
            pltpu.VMEM((SCH,), jnp.float32),
            pltpu.VMEM((LBINS_P,), jnp.int32),
            pltpu.VMEM((LBINS_P,), jnp.int32),
            pltpu.VMEM((TBLN,), jnp.int32),
            pltpu.VMEM((TBLN,), jnp.int32),
            pltpu.VMEM((REGW,), jnp.float32),
            pltpu.SemaphoreType.DMA,
        ],
    )
    return k(_sortflush_body)(idxf, vr, vi, cc)


NRUN = NW * SUB
BRUN = 32
RBW = BRUN * REC_W
QW = QROWS * NKX


def _accum_body(brec, offq, lenq, numflat, wts, csq,
                acr, aci, acw, acc, stage_a, stage_b, tailb,
                obuf_a, lbuf_a, obuf_b, lbuf_b, sem_a, sem_b, semt):
    wid = _wid()
    lane = lax.iota(jnp.int32, LANES)
    lane4 = lane * REC_W
    zerof = jnp.zeros((LANES,), jnp.float32)
    onef = jnp.ones((LANES,), jnp.float32)

    def zero_one(ac):
        def z(i, c2):
            ac[pl.ds(i * 16, 16)] = zerof
            return c2
        lax.fori_loop(0, QW // 16, z, 0)

    for ac0 in (acr, aci, acw, acc):
        zero_one(ac0)

    def load_and_fire(r, obuf, lbuf, stage, sem):
        qs = jnp.minimum(r, QS // NW - 1) * NW + wid
        pltpu.sync_copy(offq.at[qs], obuf)
        pltpu.sync_copy(lenq.at[qs], lbuf)

        def fire(runi, c2):
            off = jnp.max(plsc.load_gather(
                obuf, [jnp.full((LANES,), runi, jnp.int32)]))
            pltpu.async_copy(
                brec.at[pl.ds(pl.multiple_of(off * REC_W, 8), RBW)],
                stage.at[pl.ds(runi * RBW, RBW)], sem)
            return c2

        lax.fori_loop(0, NRUN, fire, 0)

    def drain(stage, sem):
        def d(runi, c2):
            pltpu.make_async_copy(
                brec.at[pl.ds(0, RBW)],
                stage.at[pl.ds(runi * RBW, RBW)], sem).wait()
            return c2
        lax.fori_loop(0, NRUN, d, 0)

    def addgrp(buf, base_words, lim):
        m = (lane < lim) if False else lim
        gi = jnp.full((LANES,), base_words, jnp.int32) + lane4
        rel = plsc.bitcast(plsc.load_gather(buf, [gi]), jnp.int32)
        vrv = plsc.load_gather(buf, [gi + 1])
        viv = plsc.load_gather(buf, [gi + 2])
        ccv = plsc.load_gather(buf, [gi + 3])
        hi = lax.shift_right_logical(rel, 8)
        lo = rel & jnp.int32(255)
        fidx = hi * NKX + lo
        plsc.addupdate_scatter(acr, [fidx], vrv, mask=lim)
        plsc.addupdate_scatter(aci, [fidx], viv, mask=lim)
        plsc.addupdate_scatter(acw, [fidx], onef, mask=lim)
        plsc.addupdate_scatter(acc, [fidx], ccv, mask=lim)

    def process(r, obuf, lbuf, stage):
        def grp(g, c2):
            runi = lax.shift_right_logical(g, 1)
            sub16 = (g & jnp.int32(1)) * 16
            ln = jnp.max(plsc.load_gather(
                lbuf, [jnp.full((LANES,), runi, jnp.int32)]))
            m = (sub16 + lane) < jnp.full((LANES,), jnp.minimum(ln, BRUN),
                                          jnp.int32)
            addgrp(stage, runi * RBW + sub16 * REC_W, m)
            return c2

        lax.fori_loop(0, NRUN * 2, grp, 0)

        def tail(runi, c2):
            ln = jnp.max(plsc.load_gather(
                lbuf, [jnp.full((LANES,), runi, jnp.int32)]))
            off = jnp.max(plsc.load_gather(
                obuf, [jnp.full((LANES,), runi, jnp.int32)]))
            nblk = lax.div(ln + (BRUN - 1), BRUN)

            def tblk(b, c3):
                toff = pl.multiple_of((off + b * BRUN) * REC_W, 8)
                pltpu.sync_copy(brec.at[pl.ds(toff, RBW)], tailb)
                rem = ln - b * BRUN

                def tv(g, c4):
                    m = (g * 16 + lane) < jnp.full((LANES,), rem, jnp.int32)
                    addgrp(tailb, g * 16 * REC_W, m)
                    return c4

                lax.fori_loop(0, 2, tv, 0)
                return c3

            lax.fori_loop(1, nblk, tblk, 0)
            return c2

        lax.fori_loop(0, NRUN, tail, 0)

        qs = r * NW + wid
        w0 = qs * QW
        h0 = pltpu.async_copy(acr, numflat.at[pl.ds(w0, QW)], semt)
        h1 = pltpu.async_copy(aci, numflat.at[pl.ds(NZY * NKX + w0, QW)], semt)
        h2 = pltpu.async_copy(acw, wts.at[pl.ds(w0, QW)], semt)
        h3 = pltpu.async_copy(acc, csq.at[pl.ds(w0, QW)], semt)
        for h, ac in ((h0, acr), (h1, aci), (h2, acw), (h3, acc)):
            h.wait()
            zero_one(ac)

    load_and_fire(0, obuf_a, lbuf_a, stage_a, sem_a)

    def rnd2(r2, carry):
        ra = r2 * 2
        load_and_fire(ra + 1, obuf_b, lbuf_b, stage_b, sem_b)
        drain(stage_a, sem_a)
        process(ra, obuf_a, lbuf_a, stage_a)
        load_and_fire(ra + 2, obuf_a, lbuf_a, stage_a, sem_a)
        drain(stage_b, sem_b)
        process(ra + 1, obuf_b, lbuf_b, stage_b)
        return carry

    lax.fori_loop(0, QS // NW // 2, rnd2, 0)
    drain(stage_a, sem_a)


def _accum(brec, offq, lenq):
    k = functools.partial(
        pl.kernel,
        mesh=_mesh(),
        compiler_params=_SC_PARAMS,
        out_type=[
            jax.ShapeDtypeStruct((2 * NZY * NKX,), jnp.float32),
            jax.ShapeDtypeStruct((NZY * NKX,), jnp.float32),
            jax.ShapeDtypeStruct((NZY * NKX,), jnp.float32),
        ],
        scratch_types=[
            pltpu.VMEM((QW,), jnp.float32),
            pltpu.VMEM((QW,), jnp.float32),
            pltpu.VMEM((QW,), jnp.float32),
            pltpu.VMEM((QW,), jnp.float32),
            pltpu.VMEM((NRUN * RBW,), jnp.float32),
            pltpu.VMEM((NRUN * RBW,), jnp.float32),
            pltpu.VMEM((RBW,), jnp.float32),
            pltpu.VMEM((NRUN,), jnp.int32),
            pltpu.VMEM((NRUN,), jnp.int32),
            pltpu.VMEM((NRUN,), jnp.int32),
            pltpu.VMEM((NRUN,), jnp.int32),
            pltpu.SemaphoreType.DMA,
            pltpu.SemaphoreType.DMA,
            pltpu.SemaphoreType.DMA,
        ],
    )
    return k(_accum_body)(brec, offq, lenq)


def kernel(imgs, ctf, rotMats, hwShiftAngs, numerator, weights, ctfsq):
    f = jnp.fft.rfftn(imgs, axes=(-2, -1))
    fr = jnp.real(f).astype(jnp.float32)
    fi = jnp.imag(f).astype(jnp.float32)
    ky = jnp.fft.fftfreq(D).astype(jnp.float32)
    kx = jnp.fft.rfftfreq(D).astype(jnp.float32)
    syky = hwShiftAngs[:, 0, None] * ky[None, :]
    sxkx = hwShiftAngs[:, 1, None] * kx[None, :]

    yc = (jnp.fft.fftfreq(D) * D).astype(jnp.float32)
    xc = jnp.arange(NKX, dtype=jnp.float32)
    gx = jnp.broadcast_to(xc[None, :], (D, NKX))
    gy = jnp.broadcast_to(yc[:, None], (D, NKX))
    gz = jnp.zeros((D, NKX), dtype=jnp.float32)
    grid = jnp.stack([gx, gy, gz], axis=-1)
    rot = jnp.einsum('bij,hwj->bhwi', rotMats, grid)
    neg = rot[..., 0] < 0
    rot = jnp.where(neg[..., None], -rot, rot)
    xi = jnp.round(rot[..., 0]).astype(jnp.int32)
    yi = jnp.round(rot[..., 1]).astype(jnp.int32)
    zi = jnp.round(rot[..., 2]).astype(jnp.int32)
    half = D // 2
    valid = (xi >= 0) & (xi < NKX) & (jnp.abs(yi) < half) & (jnp.abs(zi) < half)
    yi = jnp.mod(yi, D)
    zi = jnp.mod(zi, D)
    code = (zi * D + yi) * 256 + xi
    idxf = jnp.where(valid, code, SENT).reshape(NPTS)

    vr, vi, cc = _prep(fr, fi, ctf, neg.astype(jnp.float32), syky, sxkx)
    vr = vr.reshape(NPTS)
    vi = vi.reshape(NPTS)
    cc = cc.reshape(NPTS)

    brec, offt, lent = _sortflush(idxf, vr, vi, cc)

    offq = offt[:, :, :QS].reshape(NRUN, QS).T.reshape(QS, NRUN)
    lenq = lent[:, :, :QS].reshape(NRUN, QS).T.reshape(QS, NRUN)

    numflat, wtsf, csqf = _accum(brec, offq, lenq)

    new_num = numflat.reshape(2, D, D, NKX)
    new_w = wtsf.reshape(D, D, NKX)
    new_c = csqf.reshape(D, D, NKX)
    return new_num, new_w, new_c


# speedup vs baseline: 1.2741x; 1.2741x over previous
import functools

import jax
import jax.numpy as jnp
from jax import lax
from jax.experimental import pallas as pl
from jax.experimental.pallas import tpu as pltpu
from jax.experimental.pallas import tpu_sc as plsc

D = 256
NKX = D // 2 + 1
NIMG = 32
NPTS = NIMG * D * NKX
NZY = D * D
SENT = 1 << 24
QS = 1024
QROWS = 64
NW = 32
CHUNK = NPTS // NW
SUB = 4
SCH = CHUNK // SUB
LANES = 16
LBINS = (QS + 1) * LANES
LBINS_P = 16416
RS = SCH + QS + 16
TBLN = 1040
BRUN = 64

REC_W = 4
REGW = RS * REC_W
BRECW = NW * SUB * REGW + 1024

_SC_PARAMS = pltpu.CompilerParams(needs_layout_passes=False)


def _mesh():
    return plsc.VectorSubcoreMesh(core_axis_name="c", subcore_axis_name="s")


def _wid():
    return lax.axis_index("s") * 2 + lax.axis_index("c")


def _prep_body(fr_ref, fi_ref, ctf_ref, neg_ref, syky_ref, sxkx_ref,
               vr_ref, vi_ref, cc_ref):
    fr = fr_ref[0]
    fi = fi_ref[0]
    ctf = ctf_ref[0]
    ph = (-2.0 * jnp.pi) * (syky_ref[0, 0][:, None] + sxkx_ref[0, 0][None, :])
    c = jnp.cos(ph)
    s = jnp.sin(ph)
    pr = (fr * c - fi * s) * ctf
    pi = (fr * s + fi * c) * ctf
    sign = 1.0 - 2.0 * neg_ref[0]
    vr_ref[0] = pr
    vi_ref[0] = pi * sign
    cc_ref[0] = ctf * ctf


def _prep(fr, fi, ctf, negf, syky, sxkx):
    blk = pl.BlockSpec((1, D, NKX), lambda b: (b, 0, 0))
    sblk = lambda n: pl.BlockSpec((1, 1, n), lambda b: (b, 0, 0))
    return pl.pallas_call(
        _prep_body,
        grid=(NIMG,),
        in_specs=[blk, blk, blk, blk, sblk(D), sblk(NKX)],
        out_specs=[blk, blk, blk],
        out_shape=[jax.ShapeDtypeStruct((NIMG, D, NKX), jnp.float32)] * 3,
    )(fr, fi, ctf, negf, syky.reshape(NIMG, 1, D), sxkx.reshape(NIMG, 1, NKX))


def _sortflush_body(idxf, vr, vi, cc, brec, offt, lent,
                    ibuf, vbr, vbi, vbc, lhist, loff, ptab, ltab,
                    sorted_buf, sem):
    wid = _wid()
    lane = lax.iota(jnp.int32, LANES)
    zero16 = jnp.zeros((LANES,), jnp.int32)
    one16 = jnp.ones((LANES,), jnp.int32)

    def subchunk(s, carry):
        base = wid * CHUNK + s * SCH
        region = (wid * SUB + s) * RS

        pltpu.sync_copy(idxf.at[pl.ds(base, SCH)], ibuf)
        pltpu.sync_copy(vr.at[pl.ds(base, SCH)], vbr)
        pltpu.sync_copy(vi.at[pl.ds(base, SCH)], vbi)
        pltpu.sync_copy(cc.at[pl.ds(base, SCH)], vbc)

        def zero(i, c2):
            lhist[pl.ds(i * 16, 16)] = zero16
            return c2

        lax.fori_loop(0, LBINS_P // 16, zero, 0)

        def zero2(i, c2):
            ltab[pl.ds(i * 16, 16)] = zero16
            return c2

        lax.fori_loop(0, TBLN // 16, zero2, 0)

        def h1(v, c2):
            iv = ibuf[pl.ds(v * 16, 16)]
            q = lax.shift_right_logical(iv, 14)
            plsc.addupdate_scatter(lhist, [q * LANES + lane], one16)
            plsc.addupdate_scatter(ltab, [q], one16)
            return c2

        lax.fori_loop(0, SCH // 16, h1, 0)

        def pfxa(g, run):
            tot = ltab[pl.ds(g * 16, 16)]
            ptot = (tot + 1) & jnp.int32(~1)
            cs = plsc.cumsum(ptot)
            ptab[pl.ds(g * 16, 16)] = (cs - ptot) + jnp.full(
                (LANES,), run + region, jnp.int32)
            return run + jnp.max(cs)

        lax.fori_loop(0, (QS + 16) // 16, pfxa, jnp.int32(0))

        pltpu.sync_copy(ptab, offt.at[wid, s])
        pltpu.sync_copy(ltab, lent.at[wid, s])

        def pfxb(q, c2):
            h = lhist[pl.ds(q * 16, 16)]
            excl = plsc.cumsum(h) - h
            qb = plsc.load_gather(ptab, [jnp.full((LANES,), q, jnp.int32)])
            loff[pl.ds(q * 16, 16)] = excl + qb - jnp.full(
                (LANES,), region, jnp.int32)
            return c2

        lax.fori_loop(0, QS + 1, pfxb, 0)

        def p2(v, c2):
            iv = ibuf[pl.ds(v * 16, 16)]
            binc = lax.shift_right_logical(iv, 14) * LANES + lane
            pos = plsc.load_gather(loff, [binc])
            plsc.store_scatter(loff, [binc], pos + 1)
            widx = pos * REC_W
            rel = iv & jnp.int32(16383)
            plsc.store_scatter(sorted_buf, [widx],
                               plsc.bitcast(rel, jnp.float32))
            plsc.store_scatter(sorted_buf, [widx + 1], vbr[pl.ds(v * 16, 16)])
            plsc.store_scatter(sorted_buf, [widx + 2], vbi[pl.ds(v * 16, 16)])
            plsc.store_scatter(sorted_buf, [widx + 3], vbc[pl.ds(v * 16, 16)])
            return c2

        lax.fori_loop(0, SCH // 16, p2, 0)

        pltpu.sync_copy(sorted_buf, brec.at[pl.ds(region * REC_W, REGW)])
        return carry

    lax.fori_loop(0, SUB, subchunk, 0)


def _sortflush(idxf, vr, vi, cc):
    k = functools.partial(
        pl.kernel,
        mesh=_mesh(),
        compiler_params=_SC_PARAMS,
        out_type=[
            jax.ShapeDtypeStruct((BRECW,), jnp.float32),
            jax.ShapeDtypeStruct((NW, SUB, TBLN), jnp.int32),
            jax.ShapeDtypeStruct((NW, SUB, TBLN), jnp.int32),
        ],
        scratch_types=[
            pltpu.VMEM((SCH,), jnp.int32),
            pltpu.VMEM((SCH,), jnp.float32),
            pltpu.VMEM((SCH,), jnp.float32),
            pltpu.VMEM((SCH,), jnp.float32),
            pltpu.VMEM((LBINS_P,), jnp.int32),
            pltpu.VMEM((LBINS_P,), jnp.int32),
            pltpu.VMEM((TBLN,), jnp.int32),
            pltpu.VMEM((TBLN,), jnp.int32),
            pltpu.VMEM((REGW,), jnp.float32),
            pltpu.SemaphoreType.DMA,
        ],
    )
    return k(_sortflush_body)(idxf, vr, vi, cc)


NRUN = NW * SUB
BRUN = 64
RBW = BRUN * REC_W
QW = QROWS * NKX


def _accum_body(brec, offq, lenq, numflat, wts, csq,
                acr, aci, acw, acc, stage_a, stage_b, tailb,
                obuf_a, lbuf_a, obuf_b, lbuf_b, sem_a, sem_b, semt):
    wid = _wid()
    lane = lax.iota(jnp.int32, LANES)
    lane4 = lane * REC_W
    zerof = jnp.zeros((LANES,), jnp.float32)
    onef = jnp.ones((LANES,), jnp.float32)

    def zero_one(ac):
        def z(i, c2):
            ac[pl.ds(i * 16, 16)] = zerof
            return c2
        lax.fori_loop(0, QW // 16, z, 0)

    for ac0 in (acr, aci, acw, acc):
        zero_one(ac0)

    def load_and_fire(r, obuf, lbuf, stage, sem):
        qs = jnp.minimum(r, QS // NW - 1) * NW + wid
        pltpu.sync_copy(offq.at[qs], obuf)
        pltpu.sync_copy(lenq.at[qs], lbuf)

        def fire(runi, c2):
            off = jnp.max(plsc.load_gather(
                obuf, [jnp.full((LANES,), runi, jnp.int32)]))
            pltpu.async_copy(
                brec.at[pl.ds(pl.multiple_of(off * REC_W, 8), RBW)],
                stage.at[pl.ds(runi * RBW, RBW)], sem)
            return c2

        lax.fori_loop(0, NRUN, fire, 0)

    def drain(stage, sem):
        def d(runi, c2):
            pltpu.make_async_copy(
                brec.at[pl.ds(0, RBW)],
                stage.at[pl.ds(runi * RBW, RBW)], sem).wait()
            return c2
        lax.fori_loop(0, NRUN, d, 0)

    def addgrp(buf, base_words, lim):
        m = (lane < lim) if False else lim
        gi = jnp.full((LANES,), base_words, jnp.int32) + lane4
        rel = plsc.bitcast(plsc.load_gather(buf, [gi]), jnp.int32)
        vrv = plsc.load_gather(buf, [gi + 1])
        viv = plsc.load_gather(buf, [gi + 2])
        ccv = plsc.load_gather(buf, [gi + 3])
        hi = lax.shift_right_logical(rel, 8)
        lo = rel & jnp.int32(255)
        fidx = hi * NKX + lo
        plsc.addupdate_scatter(acr, [fidx], vrv, mask=lim)
        plsc.addupdate_scatter(aci, [fidx], viv, mask=lim)
        plsc.addupdate_scatter(acw, [fidx], onef, mask=lim)
        plsc.addupdate_scatter(acc, [fidx], ccv, mask=lim)

    def process(r, obuf, lbuf, stage):
        def run_one(runi, c2):
            ln = jnp.max(plsc.load_gather(
                lbuf, [jnp.full((LANES,), runi, jnp.int32)]))
            head_n = jnp.minimum(ln, BRUN)
            nvec = lax.div(head_n + 15, 16)
            limv = jnp.full((LANES,), head_n, jnp.int32)

            def vloop(g, c3):
                m = (g * 16 + lane) < limv
                addgrp(stage, runi * RBW + g * 16 * REC_W, m)
                return c3

            lax.fori_loop(0, nvec, vloop, 0)

            nblk = lax.div(ln + (BRUN - 1), BRUN)

            def tblk(b, c3):
                off = jnp.max(plsc.load_gather(
                    obuf, [jnp.full((LANES,), runi, jnp.int32)]))
                toff = pl.multiple_of((off + b * BRUN) * REC_W, 8)
                pltpu.sync_copy(brec.at[pl.ds(toff, RBW)], tailb)
                rem = jnp.full((LANES,), ln - b * BRUN, jnp.int32)

                def tv(g, c4):
                    m = (g * 16 + lane) < rem
                    addgrp(tailb, g * 16 * REC_W, m)
                    return c4

                lax.fori_loop(0, 4, tv, 0)
                return c3

            lax.fori_loop(1, nblk, tblk, 0)
            return c2

        lax.fori_loop(0, NRUN, run_one, 0)

        qs = r * NW + wid
        w0 = qs * QW
        h0 = pltpu.async_copy(acr, numflat.at[pl.ds(w0, QW)], semt)
        h1 = pltpu.async_copy(aci, numflat.at[pl.ds(NZY * NKX + w0, QW)], semt)
        h2 = pltpu.async_copy(acw, wts.at[pl.ds(w0, QW)], semt)
        h3 = pltpu.async_copy(acc, csq.at[pl.ds(w0, QW)], semt)
        for h, ac in ((h0, acr), (h1, aci), (h2, acw), (h3, acc)):
            h.wait()
            zero_one(ac)

    load_and_fire(0, obuf_a, lbuf_a, stage_a, sem_a)

    def rnd2(r2, carry):
        ra = r2 * 2
        load_and_fire(ra + 1, obuf_b, lbuf_b, stage_b, sem_b)
        drain(stage_a, sem_a)
        process(ra, obuf_a, lbuf_a, stage_a)
        load_and_fire(ra + 2, obuf_a, lbuf_a, stage_a, sem_a)
        drain(stage_b, sem_b)
        process(ra + 1, obuf_b, lbuf_b, stage_b)
        return carry

    lax.fori_loop(0, QS // NW // 2, rnd2, 0)
    drain(stage_a, sem_a)


def _accum(brec, offq, lenq):
    k = functools.partial(
        pl.kernel,
        mesh=_mesh(),
        compiler_params=_SC_PARAMS,
        out_type=[
            jax.ShapeDtypeStruct((2 * NZY * NKX,), jnp.float32),
            jax.ShapeDtypeStruct((NZY * NKX,), jnp.float32),
            jax.ShapeDtypeStruct((NZY * NKX,), jnp.float32),
        ],
        scratch_types=[
            pltpu.VMEM((QW,), jnp.float32),
            pltpu.VMEM((QW,), jnp.float32),
            pltpu.VMEM((QW,), jnp.float32),
            pltpu.VMEM((QW,), jnp.float32),
            pltpu.VMEM((NRUN * RBW,), jnp.float32),
            pltpu.VMEM((NRUN * RBW,), jnp.float32),
            pltpu.VMEM((RBW,), jnp.float32),
            pltpu.VMEM((NRUN,), jnp.int32),
            pltpu.VMEM((NRUN,), jnp.int32),
            pltpu.VMEM((NRUN,), jnp.int32),
            pltpu.VMEM((NRUN,), jnp.int32),
            pltpu.SemaphoreType.DMA,
            pltpu.SemaphoreType.DMA,
            pltpu.SemaphoreType.DMA,
        ],
    )
    return k(_accum_body)(brec, offq, lenq)


def kernel(imgs, ctf, rotMats, hwShiftAngs, numerator, weights, ctfsq):
    f = jnp.fft.rfftn(imgs, axes=(-2, -1))
    fr = jnp.real(f).astype(jnp.float32)
    fi = jnp.imag(f).astype(jnp.float32)
    ky = jnp.fft.fftfreq(D).astype(jnp.float32)
    kx = jnp.fft.rfftfreq(D).astype(jnp.float32)
    syky = hwShiftAngs[:, 0, None] * ky[None, :]
    sxkx = hwShiftAngs[:, 1, None] * kx[None, :]

    yc = (jnp.fft.fftfreq(D) * D).astype(jnp.float32)
    xc = jnp.arange(NKX, dtype=jnp.float32)
    gx = jnp.broadcast_to(xc[None, :], (D, NKX))
    gy = jnp.broadcast_to(yc[:, None], (D, NKX))
    gz = jnp.zeros((D, NKX), dtype=jnp.float32)
    grid = jnp.stack([gx, gy, gz], axis=-1)
    rot = jnp.einsum('bij,hwj->bhwi', rotMats, grid)
    neg = rot[..., 0] < 0
    rot = jnp.where(neg[..., None], -rot, rot)
    xi = jnp.round(rot[..., 0]).astype(jnp.int32)
    yi = jnp.round(rot[..., 1]).astype(jnp.int32)
    zi = jnp.round(rot[..., 2]).astype(jnp.int32)
    half = D // 2
    valid = (xi >= 0) & (xi < NKX) & (jnp.abs(yi) < half) & (jnp.abs(zi) < half)
    yi = jnp.mod(yi, D)
    zi = jnp.mod(zi, D)
    code = (zi * D + yi) * 256 + xi
    idxf = jnp.where(valid, code, SENT).reshape(NPTS)

    vr, vi, cc = _prep(fr, fi, ctf, neg.astype(jnp.float32), syky, sxkx)
    vr = vr.reshape(NPTS)
    vi = vi.reshape(NPTS)
    cc = cc.reshape(NPTS)

    brec, offt, lent = _sortflush(idxf, vr, vi, cc)

    offq = offt[:, :, :QS].reshape(NRUN, QS).T.reshape(QS, NRUN)
    lenq = lent[:, :, :QS].reshape(NRUN, QS).T.reshape(QS, NRUN)

    numflat, wtsf, csqf = _accum(brec, offq, lenq)

    new_num = numflat.reshape(2, D, D, NKX)
    new_w = wtsf.reshape(D, D, NKX)
    new_c = csqf.reshape(D, D, NKX)
    return new_num, new_w, new_c


# speedup vs baseline: 1.3426x; 1.0537x over previous
import functools

import jax
import jax.numpy as jnp
from jax import lax
from jax.experimental import pallas as pl
from jax.experimental.pallas import tpu as pltpu
from jax.experimental.pallas import tpu_sc as plsc

D = 256
NKX = D // 2 + 1
NIMG = 32
NPTS = NIMG * D * NKX
NZY = D * D
SENT = 1 << 24
QS = 1024
QROWS = 64
NW = 32
CHUNK = NPTS // NW
SUB = 4
SCH = CHUNK // SUB
LANES = 16
LBINS = (QS + 1) * LANES
LBINS_P = 16416
RS = SCH + QS + 16
TBLN = 1040
BRUN = 64

REC_W = 4
REGW = RS * REC_W
BRECW = NW * SUB * REGW + 1024

_SC_PARAMS = pltpu.CompilerParams(needs_layout_passes=False)


def _mesh():
    return plsc.VectorSubcoreMesh(core_axis_name="c", subcore_axis_name="s")


def _wid():
    return lax.axis_index("s") * 2 + lax.axis_index("c")


def _prep_body(img_ref, mr_ref, mi_ref, er_ref, ei_ref,
               ctf_ref, neg_ref, syky_ref, sxkx_ref,
               vr_ref, vi_ref, cc_ref):
    img = img_ref[0]
    mr = mr_ref[...]
    mi = mi_ref[...]
    er = er_ref[...]
    ei = ei_ref[...]
    tr = jnp.dot(img, mr, preferred_element_type=jnp.float32)
    ti = jnp.dot(img, mi, preferred_element_type=jnp.float32)
    fr = (jnp.dot(er, tr, preferred_element_type=jnp.float32)
          - jnp.dot(ei, ti, preferred_element_type=jnp.float32))
    fi = (jnp.dot(er, ti, preferred_element_type=jnp.float32)
          + jnp.dot(ei, tr, preferred_element_type=jnp.float32))
    ctf = ctf_ref[0]
    ph = (-2.0 * jnp.pi) * (syky_ref[0, 0][:, None] + sxkx_ref[0, 0][None, :])
    c = jnp.cos(ph)
    s = jnp.sin(ph)
    pr = (fr * c - fi * s) * ctf
    pi = (fr * s + fi * c) * ctf
    sign = 1.0 - 2.0 * neg_ref[0]
    vr_ref[0] = pr
    vi_ref[0] = pi * sign
    cc_ref[0] = ctf * ctf


def _prep(imgs, mr, mi, er, ei, ctf, negf, syky, sxkx):
    blk = pl.BlockSpec((1, D, NKX), lambda b: (b, 0, 0))
    iblk = pl.BlockSpec((1, D, D), lambda b: (b, 0, 0))
    cblk = lambda r, c: pl.BlockSpec((r, c), lambda b: (0, 0))
    sblk = lambda n: pl.BlockSpec((1, 1, n), lambda b: (b, 0, 0))
    return pl.pallas_call(
        _prep_body,
        grid=(NIMG,),
        in_specs=[iblk, cblk(D, NKX), cblk(D, NKX), cblk(D, D), cblk(D, D),
                  blk, blk, sblk(D), sblk(NKX)],
        out_specs=[blk, blk, blk],
        out_shape=[jax.ShapeDtypeStruct((NIMG, D, NKX), jnp.float32)] * 3,
    )(imgs, mr, mi, er, ei, ctf, negf,
      syky.reshape(NIMG, 1, D), sxkx.reshape(NIMG, 1, NKX))


def _sortflush_body(idxf, vr, vi, cc, brec, offt, lent,
                    ibuf, vbr, vbi, vbc, lhist, loff, ptab, ltab,
                    sorted_buf, sem):
    wid = _wid()
    lane = lax.iota(jnp.int32, LANES)
    zero16 = jnp.zeros((LANES,), jnp.int32)
    one16 = jnp.ones((LANES,), jnp.int32)

    def subchunk(s, carry):
        base = wid * CHUNK + s * SCH
        region = (wid * SUB + s) * RS

        pltpu.sync_copy(idxf.at[pl.ds(base, SCH)], ibuf)
        pltpu.sync_copy(vr.at[pl.ds(base, SCH)], vbr)
        pltpu.sync_copy(vi.at[pl.ds(base, SCH)], vbi)
        pltpu.sync_copy(cc.at[pl.ds(base, SCH)], vbc)

        def zero(i, c2):
            lhist[pl.ds(i * 16, 16)] = zero16
            return c2

        lax.fori_loop(0, LBINS_P // 16, zero, 0)

        def zero2(i, c2):
            ltab[pl.ds(i * 16, 16)] = zero16
            return c2

        lax.fori_loop(0, TBLN // 16, zero2, 0)

        def h1(v, c2):
            iv = ibuf[pl.ds(v * 16, 16)]
            q = lax.shift_right_logical(iv, 14)
            plsc.addupdate_scatter(lhist, [q * LANES + lane], one16)
            plsc.addupdate_scatter(ltab, [q], one16)
            return c2

        lax.fori_loop(0, SCH // 16, h1, 0)

        def pfxa(g, run):
            tot = ltab[pl.ds(g * 16, 16)]
            ptot = (tot + 1) & jnp.int32(~1)
            cs = plsc.cumsum(ptot)
            ptab[pl.ds(g * 16, 16)] = (cs - ptot) + jnp.full(
                (LANES,), run + region, jnp.int32)
            return run + jnp.max(cs)

        lax.fori_loop(0, (QS + 16) // 16, pfxa, jnp.int32(0))

        pltpu.sync_copy(ptab, offt.at[wid, s])
        pltpu.sync_copy(ltab, lent.at[wid, s])

        def pfxb(q, c2):
            h = lhist[pl.ds(q * 16, 16)]
            excl = plsc.cumsum(h) - h
            qb = plsc.load_gather(ptab, [jnp.full((LANES,), q, jnp.int32)])
            loff[pl.ds(q * 16, 16)] = excl + qb - jnp.full(
                (LANES,), region, jnp.int32)
            return c2

        lax.fori_loop(0, QS + 1, pfxb, 0)

        def p2(v, c2):
            iv = ibuf[pl.ds(v * 16, 16)]
            binc = lax.shift_right_logical(iv, 14) * LANES + lane
            pos = plsc.load_gather(loff, [binc])
            plsc.store_scatter(loff, [binc], pos + 1)
            widx = pos * REC_W
            rel = iv & jnp.int32(16383)
            plsc.store_scatter(sorted_buf, [widx],
                               plsc.bitcast(rel, jnp.float32))
            plsc.store_scatter(sorted_buf, [widx + 1], vbr[pl.ds(v * 16, 16)])
            plsc.store_scatter(sorted_buf, [widx + 2], vbi[pl.ds(v * 16, 16)])
            plsc.store_scatter(sorted_buf, [widx + 3], vbc[pl.ds(v * 16, 16)])
            return c2

        lax.fori_loop(0, SCH // 16, p2, 0)

        pltpu.sync_copy(sorted_buf, brec.at[pl.ds(region * REC_W, REGW)])
        return carry

    lax.fori_loop(0, SUB, subchunk, 0)


def _sortflush(idxf, vr, vi, cc):
    k = functools.partial(
        pl.kernel,
        mesh=_mesh(),
        compiler_params=_SC_PARAMS,
        out_type=[
            jax.ShapeDtypeStruct((BRECW,), jnp.float32),
            jax.ShapeDtypeStruct((NW, SUB, TBLN), jnp.int32),
            jax.ShapeDtypeStruct((NW, SUB, TBLN), jnp.int32),
        ],
        scratch_types=[
            pltpu.VMEM((SCH,), jnp.int32),
            pltpu.VMEM((SCH,), jnp.float32),
            pltpu.VMEM((SCH,), jnp.float32),
            pltpu.VMEM((SCH,), jnp.float32),
            pltpu.VMEM((LBINS_P,), jnp.int32),
            pltpu.VMEM((LBINS_P,), jnp.int32),
            pltpu.VMEM((TBLN,), jnp.int32),
            pltpu.VMEM((TBLN,), jnp.int32),
            pltpu.VMEM((REGW,), jnp.float32),
            pltpu.SemaphoreType.DMA,
        ],
    )
    return k(_sortflush_body)(idxf, vr, vi, cc)


NRUN = NW * SUB
BRUN = 64
RBW = BRUN * REC_W
QW = QROWS * NKX


def _accum_body(brec, offq, lenq, numflat, wts, csq,
                acr, aci, acw, acc, stage_a, stage_b, tailb,
                obuf_a, lbuf_a, obuf_b, lbuf_b, sem_a, sem_b, semt):
    wid = _wid()
    lane = lax.iota(jnp.int32, LANES)
    lane4 = lane * REC_W
    zerof = jnp.zeros((LANES,), jnp.float32)
    onef = jnp.ones((LANES,), jnp.float32)

    def zero_one(ac):
        def z(i, c2):
            ac[pl.ds(i * 16, 16)] = zerof
            return c2
        lax.fori_loop(0, QW // 16, z, 0)

    for ac0 in (acr, aci, acw, acc):
        zero_one(ac0)

    def load_and_fire(r, obuf, lbuf, stage, sem):
        qs = jnp.minimum(r, QS // NW - 1) * NW + wid
        pltpu.sync_copy(offq.at[qs], obuf)
        pltpu.sync_copy(lenq.at[qs], lbuf)

        def fire(runi, c2):
            off = jnp.max(plsc.load_gather(
                obuf, [jnp.full((LANES,), runi, jnp.int32)]))
            pltpu.async_copy(
                brec.at[pl.ds(pl.multiple_of(off * REC_W, 8), RBW)],
                stage.at[pl.ds(runi * RBW, RBW)], sem)
            return c2

        lax.fori_loop(0, NRUN, fire, 0)

    def drain(stage, sem):
        def d(runi, c2):
            pltpu.make_async_copy(
                brec.at[pl.ds(0, RBW)],
                stage.at[pl.ds(runi * RBW, RBW)], sem).wait()
            return c2
        lax.fori_loop(0, NRUN, d, 0)

    def addgrp(buf, base_words, lim):
        m = (lane < lim) if False else lim
        gi = jnp.full((LANES,), base_words, jnp.int32) + lane4
        rel = plsc.bitcast(plsc.load_gather(buf, [gi]), jnp.int32)
        vrv = plsc.load_gather(buf, [gi + 1])
        viv = plsc.load_gather(buf, [gi + 2])
        ccv = plsc.load_gather(buf, [gi + 3])
        hi = lax.shift_right_logical(rel, 8)
        lo = rel & jnp.int32(255)
        fidx = hi * NKX + lo
        plsc.addupdate_scatter(acr, [fidx], vrv, mask=lim)
        plsc.addupdate_scatter(aci, [fidx], viv, mask=lim)
        plsc.addupdate_scatter(acw, [fidx], onef, mask=lim)
        plsc.addupdate_scatter(acc, [fidx], ccv, mask=lim)

    def process(r, obuf, lbuf, stage):
        def run_one(runi, c2):
            ln = jnp.max(plsc.load_gather(
                lbuf, [jnp.full((LANES,), runi, jnp.int32)]))
            head_n = jnp.minimum(ln, BRUN)
            nvec = lax.div(head_n + 15, 16)
            limv = jnp.full((LANES,), head_n, jnp.int32)

            def vloop(g, c3):
                m = (g * 16 + lane) < limv
                addgrp(stage, runi * RBW + g * 16 * REC_W, m)
                return c3

            lax.fori_loop(0, nvec, vloop, 0)

            nblk = lax.div(ln + (BRUN - 1), BRUN)

            def tblk(b, c3):
                off = jnp.max(plsc.load_gather(
                    obuf, [jnp.full((LANES,), runi, jnp.int32)]))
                toff = pl.multiple_of((off + b * BRUN) * REC_W, 8)
                pltpu.sync_copy(brec.at[pl.ds(toff, RBW)], tailb)
                rem = jnp.full((LANES,), ln - b * BRUN, jnp.int32)

                def tv(g, c4):
                    m = (g * 16 + lane) < rem
                    addgrp(tailb, g * 16 * REC_W, m)
                    return c4

                lax.fori_loop(0, 4, tv, 0)
                return c3

            lax.fori_loop(1, nblk, tblk, 0)
            return c2

        lax.fori_loop(0, NRUN, run_one, 0)

        qs = r * NW + wid
        w0 = qs * QW
        h0 = pltpu.async_copy(acr, numflat.at[pl.ds(w0, QW)], semt)
        h1 = pltpu.async_copy(aci, numflat.at[pl.ds(NZY * NKX + w0, QW)], semt)
        h2 = pltpu.async_copy(acw, wts.at[pl.ds(w0, QW)], semt)
        h3 = pltpu.async_copy(acc, csq.at[pl.ds(w0, QW)], semt)
        for h, ac in ((h0, acr), (h1, aci), (h2, acw), (h3, acc)):
            h.wait()
            zero_one(ac)

    load_and_fire(0, obuf_a, lbuf_a, stage_a, sem_a)

    def rnd2(r2, carry):
        ra = r2 * 2
        load_and_fire(ra + 1, obuf_b, lbuf_b, stage_b, sem_b)
        drain(stage_a, sem_a)
        process(ra, obuf_a, lbuf_a, stage_a)
        load_and_fire(ra + 2, obuf_a, lbuf_a, stage_a, sem_a)
        drain(stage_b, sem_b)
        process(ra + 1, obuf_b, lbuf_b, stage_b)
        return carry

    lax.fori_loop(0, QS // NW // 2, rnd2, 0)
    drain(stage_a, sem_a)


def _accum(brec, offq, lenq):
    k = functools.partial(
        pl.kernel,
        mesh=_mesh(),
        compiler_params=_SC_PARAMS,
        out_type=[
            jax.ShapeDtypeStruct((2 * NZY * NKX,), jnp.float32),
            jax.ShapeDtypeStruct((NZY * NKX,), jnp.float32),
            jax.ShapeDtypeStruct((NZY * NKX,), jnp.float32),
        ],
        scratch_types=[
            pltpu.VMEM((QW,), jnp.float32),
            pltpu.VMEM((QW,), jnp.float32),
            pltpu.VMEM((QW,), jnp.float32),
            pltpu.VMEM((QW,), jnp.float32),
            pltpu.VMEM((NRUN * RBW,), jnp.float32),
            pltpu.VMEM((NRUN * RBW,), jnp.float32),
            pltpu.VMEM((RBW,), jnp.float32),
            pltpu.VMEM((NRUN,), jnp.int32),
            pltpu.VMEM((NRUN,), jnp.int32),
            pltpu.VMEM((NRUN,), jnp.int32),
            pltpu.VMEM((NRUN,), jnp.int32),
            pltpu.SemaphoreType.DMA,
            pltpu.SemaphoreType.DMA,
            pltpu.SemaphoreType.DMA,
        ],
    )
    return k(_accum_body)(brec, offq, lenq)


def kernel(imgs, ctf, rotMats, hwShiftAngs, numerator, weights, ctfsq):
    hh = jnp.arange(D, dtype=jnp.float32)
    ang1 = (-2.0 * jnp.pi / D) * jnp.outer(hh, hh[:NKX])
    mr, mi = jnp.cos(ang1), jnp.sin(ang1)
    ang2 = (-2.0 * jnp.pi / D) * jnp.outer(hh, hh)
    er, ei = jnp.cos(ang2), jnp.sin(ang2)
    ky = jnp.fft.fftfreq(D).astype(jnp.float32)
    kx = jnp.fft.rfftfreq(D).astype(jnp.float32)
    syky = hwShiftAngs[:, 0, None] * ky[None, :]
    sxkx = hwShiftAngs[:, 1, None] * kx[None, :]

    yc = (jnp.fft.fftfreq(D) * D).astype(jnp.float32)
    xc = jnp.arange(NKX, dtype=jnp.float32)
    gx = jnp.broadcast_to(xc[None, :], (D, NKX))
    gy = jnp.broadcast_to(yc[:, None], (D, NKX))
    gz = jnp.zeros((D, NKX), dtype=jnp.float32)
    grid = jnp.stack([gx, gy, gz], axis=-1)
    rot = jnp.einsum('bij,hwj->bhwi', rotMats, grid)
    neg = rot[..., 0] < 0
    rot = jnp.where(neg[..., None], -rot, rot)
    xi = jnp.round(rot[..., 0]).astype(jnp.int32)
    yi = jnp.round(rot[..., 1]).astype(jnp.int32)
    zi = jnp.round(rot[..., 2]).astype(jnp.int32)
    half = D // 2
    valid = (xi >= 0) & (xi < NKX) & (jnp.abs(yi) < half) & (jnp.abs(zi) < half)
    yi = jnp.mod(yi, D)
    zi = jnp.mod(zi, D)
    code = (zi * D + yi) * 256 + xi
    idxf = jnp.where(valid, code, SENT).reshape(NPTS)

    vr, vi, cc = _prep(imgs, mr, mi, er, ei, ctf,
                       neg.astype(jnp.float32), syky, sxkx)
    vr = vr.reshape(NPTS)
    vi = vi.reshape(NPTS)
    cc = cc.reshape(NPTS)

    brec, offt, lent = _sortflush(idxf, vr, vi, cc)

    offq = offt[:, :, :QS].reshape(NRUN, QS).T.reshape(QS, NRUN)
    lenq = lent[:, :, :QS].reshape(NRUN, QS).T.reshape(QS, NRUN)

    numflat, wtsf, csqf = _accum(brec, offq, lenq)

    new_num = numflat.reshape(2, D, D, NKX)
    new_w = wtsf.reshape(D, D, NKX)
    new_c = csqf.reshape(D, D, NKX)
    return new_num, new_w, new_c
